# baseline (speedup 1.0000x reference)
import functools

import jax
import jax.numpy as jnp
from jax import lax
from jax.experimental import pallas as pl
from jax.experimental.pallas import tpu as pltpu
from jax.experimental.pallas import tpu_sc as plsc

NC = 2
NS = 16
NW = NC * NS
CH = 128
KR = 16
NB = 8
ZQ = 32


def _sc_mesh():
    return plsc.VectorSubcoreMesh(core_axis_name="c", subcore_axis_name="s")


def _make_deg_kernel(r_pad, n_deg):
    rpw = r_pad // NW
    n_outer = rpw // KR
    zrows = n_deg // NS

    @functools.partial(
        pl.kernel,
        out_type=jax.ShapeDtypeStruct((NC * n_deg,), jnp.float32),
        mesh=_sc_mesh(),
        scratch_types=[
            pltpu.VMEM((KR, CH), jnp.int32),
            pltpu.VMEM((KR, CH), jnp.int32),
            pltpu.VMEM((CH,), jnp.float32),
            pltpu.VMEM((zrows,), jnp.float32),
            pltpu.VMEM_SHARED((n_deg,), jnp.float32),
            pltpu.SemaphoreType.DMA,
            pltpu.SemaphoreType.DMA,
        ],
    )
    def deg_kernel(dst_hbm, out_hbm, didxA, didxB, ones_v, zv, acc,
                   semA, semB):
        c = lax.axis_index("c")
        s = lax.axis_index("s")
        wid = s * NC + c

        for i in range(CH // 16):
            ones_v[pl.ds(i * 16, 16)] = jnp.ones((16,), jnp.float32)

        def zfill(i, _):
            zv[pl.ds(i * 16, 16)] = jnp.zeros((16,), jnp.float32)
            return 0
        lax.fori_loop(0, zrows // 16, zfill, 0)
        pltpu.sync_copy(zv, acc.at[pl.ds(s * zrows, zrows)])
        plsc.subcore_barrier()

        row0 = wid * rpw

        def load_idx(g, dbuf, sem):
            pltpu.async_copy(dst_hbm.at[pl.ds(row0 + g * KR, KR)], dbuf, sem)

        def wait_idx(dbuf, sem):
            pltpu.make_async_copy(dst_hbm.at[pl.ds(row0, KR)], dbuf, sem).wait()

        def process(didx):
            for jj in range(KR):
                pltpu.sync_copy(ones_v, acc.at[didx.at[jj]], add=True)

        load_idx(0, didxA, semA)
        load_idx(1, didxB, semB)

        def outer(p, _):
            g0 = 2 * p
            wait_idx(didxA, semA)
            process(didxA)

            @pl.when(g0 + 2 < n_outer)
            def _():
                load_idx(g0 + 2, didxA, semA)

            wait_idx(didxB, semB)
            process(didxB)

            @pl.when(g0 + 3 < n_outer)
            def _():
                load_idx(g0 + 3, didxB, semB)
            return 0
        lax.fori_loop(0, n_outer // 2, outer, 0)

        plsc.subcore_barrier()
        pltpu.sync_copy(acc.at[pl.ds(s * zrows, zrows)], zv)
        pltpu.sync_copy(zv, out_hbm.at[pl.ds(c * n_deg + s * zrows, zrows)])

    return deg_kernel


def _make_agg_kernel(n, r_pad, n_acc, f):
    rpw = r_pad // NW
    n_outer = rpw // KR
    zrows = n_acc // NS
    zb_rows = zrows // ZQ

    @functools.partial(
        pl.kernel,
        out_type=jax.ShapeDtypeStruct((NC, n_acc, f), jnp.float32),
        mesh=_sc_mesh(),
        compiler_params=pltpu.CompilerParams(use_tc_tiling_on_sc=False),
        scratch_types=[
            pltpu.VMEM((KR, CH), jnp.int32),
            pltpu.VMEM((KR, CH), jnp.int32),
            pltpu.VMEM((KR, CH), jnp.int32),
            pltpu.VMEM((KR, CH), jnp.int32),
        ] + [pltpu.VMEM((CH, f), jnp.float32)] * NB
          + [pltpu.VMEM((zb_rows, f), jnp.float32),
             pltpu.VMEM_SHARED((n_acc, f), jnp.float32)]
          + [pltpu.SemaphoreType.DMA] * (NB + 2),
    )
    def agg_kernel(tab_hbm, src_hbm, dst_hbm, out_hbm,
                   sidxA, didxA, sidxB, didxB, *rest):
        gbs = rest[:NB]
        zb = rest[NB]
        acc = rest[NB + 1]
        sems = rest[NB + 2:NB + 2 + NB]
        semA, semB = rest[NB + 2 + NB:]
        c = lax.axis_index("c")
        s = lax.axis_index("s")
        wid = s * NC + c

        def zfill(i, _):
            zb[i] = jnp.zeros((16,), jnp.float32)
            return 0
        lax.fori_loop(0, zb_rows, zfill, 0)
        for q in range(ZQ):
            pltpu.sync_copy(zb, acc.at[pl.ds(s * zrows + q * zb_rows, zb_rows)])
        plsc.subcore_barrier()

        row0 = wid * rpw

        def load_idx(g, sbuf, dbuf, sem):
            rb = row0 + g * KR
            pltpu.async_copy(src_hbm.at[pl.ds(rb, KR)], sbuf, sem)
            pltpu.async_copy(dst_hbm.at[pl.ds(rb, KR)], dbuf, sem)

        def wait_idx(sbuf, dbuf, sem):
            pltpu.make_async_copy(src_hbm.at[pl.ds(row0, KR)], sbuf, sem).wait()
            pltpu.make_async_copy(dst_hbm.at[pl.ds(row0, KR)], dbuf, sem).wait()

        def process(sidx, didx):
            handles = [
                pltpu.async_copy(tab_hbm.at[sidx.at[jj]], gbs[jj % NB],
                                 sems[jj % NB])
                for jj in range(NB)
            ]
            for jj in range(KR):
                handles[jj].wait()
                pltpu.sync_copy(gbs[jj % NB], acc.at[didx.at[jj]], add=True)
                if jj + NB < KR:
                    handles.append(
                        pltpu.async_copy(tab_hbm.at[sidx.at[jj + NB]],
                                         gbs[(jj + NB) % NB],
                                         sems[(jj + NB) % NB]))

        load_idx(0, sidxA, didxA, semA)
        load_idx(1, sidxB, didxB, semB)

        def outer(p, _):
            g0 = 2 * p
            wait_idx(sidxA, didxA, semA)
            process(sidxA, didxA)

            @pl.when(g0 + 2 < n_outer)
            def _():
                load_idx(g0 + 2, sidxA, didxA, semA)

            wait_idx(sidxB, didxB, semB)
            process(sidxB, didxB)

            @pl.when(g0 + 3 < n_outer)
            def _():
                load_idx(g0 + 3, sidxB, didxB, semB)
            return 0
        lax.fori_loop(0, n_outer // 2, outer, 0)

        plsc.subcore_barrier()
        pltpu.sync_copy(acc.at[pl.ds(s * zrows, zrows)],
                        out_hbm.at[c, pl.ds(s * zrows, zrows)])

    return agg_kernel


def _round_up(a, b):
    return (a + b - 1) // b * b


def kernel(x, edge_index, W1, b1, W2, b2):
    n, f_in = x.shape
    e = edge_index.shape[1]
    hid = W1.shape[1]
    c_out = W2.shape[1]
    assert n % NS == 0 and hid == 16

    r_pad = _round_up(-(-e // CH), NW * KR * 2)
    e_pad = r_pad * CH
    pad = e_pad - e
    src2d = jnp.concatenate(
        [edge_index[0], jnp.zeros((pad,), jnp.int32)]).reshape(r_pad, CH)
    dst2d = jnp.concatenate(
        [edge_index[1], jnp.full((pad,), n, jnp.int32)]).reshape(r_pad, CH)

    n_deg = _round_up(n + 1, NS * 16)
    n_acc = _round_up(n + 1, NS * ZQ)

    deg_kernel = _make_deg_kernel(r_pad, n_deg)
    agg_kernel = _make_agg_kernel(n, r_pad, n_acc, hid)

    deg_parts = deg_kernel(dst2d).reshape(NC, n_deg)
    deg_col = (deg_parts[0, :n] + deg_parts[1, :n]).reshape(n, 1)

    bn = 2000
    grid = (n // bn,)

    def k2_body(deg_ref, x_ref, w1_ref, hs_ref, dinv_ref):
        dinv = lax.rsqrt(deg_ref[...] + 1.0)
        h = jnp.dot(x_ref[...], w1_ref[...],
                    preferred_element_type=jnp.float32)
        hs_ref[...] = h * dinv
        dinv_ref[...] = dinv

    hs, dinv = pl.pallas_call(
        k2_body,
        grid=grid,
        in_specs=[
            pl.BlockSpec((bn, 1), lambda i: (i, 0)),
            pl.BlockSpec((bn, f_in), lambda i: (i, 0)),
            pl.BlockSpec((f_in, hid), lambda i: (0, 0)),
        ],
        out_specs=[
            pl.BlockSpec((bn, hid), lambda i: (i, 0)),
            pl.BlockSpec((bn, 1), lambda i: (i, 0)),
        ],
        out_shape=[
            jax.ShapeDtypeStruct((n, hid), jnp.float32),
            jax.ShapeDtypeStruct((n, 1), jnp.float32),
        ],
    )(deg_col, x, W1)

    agg1 = agg_kernel(hs, src2d, dst2d)

    def k4_body(a_ref, hs_ref, dinv_ref, b1_ref, rs_ref):
        t = (a_ref[0] + a_ref[1] + hs_ref[...]) * dinv_ref[...] + b1_ref[...]
        rs_ref[...] = jnp.maximum(t, 0.0) * dinv_ref[...]

    rs = pl.pallas_call(
        k4_body,
        grid=grid,
        in_specs=[
            pl.BlockSpec((NC, bn, hid), lambda i: (0, i, 0)),
            pl.BlockSpec((bn, hid), lambda i: (i, 0)),
            pl.BlockSpec((bn, 1), lambda i: (i, 0)),
            pl.BlockSpec((1, hid), lambda i: (0, 0)),
        ],
        out_specs=pl.BlockSpec((bn, hid), lambda i: (i, 0)),
        out_shape=jax.ShapeDtypeStruct((n, hid), jnp.float32),
    )(agg1, hs, dinv, b1.reshape(1, hid))

    agg2 = agg_kernel(rs, src2d, dst2d)

    def k6_body(a_ref, rs_ref, dinv_ref, w2_ref, b2_ref, out_ref):
        t = (a_ref[0] + a_ref[1] + rs_ref[...]) * dinv_ref[...]
        out_ref[...] = jnp.dot(t, w2_ref[...],
                               preferred_element_type=jnp.float32) + b2_ref[...]

    out = pl.pallas_call(
        k6_body,
        grid=grid,
        in_specs=[
            pl.BlockSpec((NC, bn, hid), lambda i: (0, i, 0)),
            pl.BlockSpec((bn, hid), lambda i: (i, 0)),
            pl.BlockSpec((bn, 1), lambda i: (i, 0)),
            pl.BlockSpec((hid, c_out), lambda i: (0, 0)),
            pl.BlockSpec((1, c_out), lambda i: (0, 0)),
        ],
        out_specs=pl.BlockSpec((bn, c_out), lambda i: (i, 0)),
        out_shape=jax.ShapeDtypeStruct((n, c_out), jnp.float32),
    )(agg2, rs, dinv, W2, b2.reshape(1, c_out))

    return out


# speedup vs baseline: 1.4677x; 1.4677x over previous
import functools

import jax
import jax.numpy as jnp
from jax import lax
from jax.experimental import pallas as pl
from jax.experimental.pallas import tpu as pltpu
from jax.experimental.pallas import tpu_sc as plsc

NC = 2
NS = 16
NW = NC * NS
CH = 128
KR = 16
NB = 8
ZQ = 32


def _sc_mesh():
    return plsc.VectorSubcoreMesh(core_axis_name="c", subcore_axis_name="s")


def _make_deg_kernel(r_pad, n_deg):
    rpw = r_pad // NW
    n_outer = rpw // KR
    zrows = n_deg // NS

    @functools.partial(
        pl.kernel,
        out_type=jax.ShapeDtypeStruct((NC * n_deg,), jnp.float32),
        mesh=_sc_mesh(),
        scratch_types=[
            pltpu.VMEM((KR, CH), jnp.int32),
            pltpu.VMEM((CH,), jnp.float32),
            pltpu.VMEM((zrows,), jnp.float32),
            pltpu.VMEM_SHARED((n_deg,), jnp.float32),
        ],
    )
    def deg_kernel(dst_hbm, out_hbm, didx, ones_v, zv, acc):
        c = lax.axis_index("c")
        s = lax.axis_index("s")
        wid = s * NC + c

        for i in range(CH // 16):
            ones_v[pl.ds(i * 16, 16)] = jnp.ones((16,), jnp.float32)

        def zfill(i, _):
            zv[pl.ds(i * 16, 16)] = jnp.zeros((16,), jnp.float32)
            return 0
        lax.fori_loop(0, zrows // 16, zfill, 0)
        pltpu.sync_copy(zv, acc.at[pl.ds(s * zrows, zrows)])
        plsc.subcore_barrier()

        row0 = wid * rpw

        def outer(g, _):
            rb = row0 + g * KR
            pltpu.sync_copy(dst_hbm.at[pl.ds(rb, KR)], didx)
            for jj in range(KR):
                pltpu.sync_copy(ones_v, acc.at[didx.at[jj]], add=True)
            return 0
        lax.fori_loop(0, n_outer, outer, 0)

        plsc.subcore_barrier()
        pltpu.sync_copy(acc.at[pl.ds(s * zrows, zrows)], zv)
        pltpu.sync_copy(zv, out_hbm.at[pl.ds(c * n_deg + s * zrows, zrows)])

    return deg_kernel


def _make_agg_kernel(n, r_pad, n_acc, f):
    rpw = r_pad // NW
    n_outer = rpw // KR
    zrows = n_acc // NS
    zb_rows = zrows // ZQ

    @functools.partial(
        pl.kernel,
        out_type=jax.ShapeDtypeStruct((NC, n_acc, f), jnp.float32),
        mesh=_sc_mesh(),
        compiler_params=pltpu.CompilerParams(use_tc_tiling_on_sc=False),
        scratch_types=[
            pltpu.VMEM((KR, CH), jnp.int32),
            pltpu.VMEM((KR, CH), jnp.int32),
        ] + [pltpu.VMEM((CH, f), jnp.float32)] * NB
          + [pltpu.VMEM((zb_rows, f), jnp.float32),
             pltpu.VMEM_SHARED((n_acc, f), jnp.float32)]
          + [pltpu.SemaphoreType.DMA] * NB,
    )
    def agg_kernel(tab_hbm, src_hbm, dst_hbm, out_hbm, sidx, didx, *rest):
        gbs = rest[:NB]
        zb = rest[NB]
        acc = rest[NB + 1]
        sems = rest[NB + 2:]
        c = lax.axis_index("c")
        s = lax.axis_index("s")
        wid = s * NC + c

        def zfill(i, _):
            zb[i] = jnp.zeros((16,), jnp.float32)
            return 0
        lax.fori_loop(0, zb_rows, zfill, 0)
        for q in range(ZQ):
            pltpu.sync_copy(zb, acc.at[pl.ds(s * zrows + q * zb_rows, zb_rows)])
        plsc.subcore_barrier()

        row0 = wid * rpw

        def outer(g, _):
            rb = row0 + g * KR
            pltpu.sync_copy(src_hbm.at[pl.ds(rb, KR)], sidx)
            pltpu.sync_copy(dst_hbm.at[pl.ds(rb, KR)], didx)
            handles = [
                pltpu.async_copy(tab_hbm.at[sidx.at[jj]], gbs[jj % NB],
                                 sems[jj % NB])
                for jj in range(NB)
            ]
            for jj in range(KR):
                handles[jj].wait()
                pltpu.sync_copy(gbs[jj % NB], acc.at[didx.at[jj]], add=True)
                if jj + NB < KR:
                    handles.append(
                        pltpu.async_copy(tab_hbm.at[sidx.at[jj + NB]],
                                         gbs[(jj + NB) % NB],
                                         sems[(jj + NB) % NB]))
            return 0
        lax.fori_loop(0, n_outer, outer, 0)

        plsc.subcore_barrier()
        pltpu.sync_copy(acc.at[pl.ds(s * zrows, zrows)],
                        out_hbm.at[c, pl.ds(s * zrows, zrows)])

    return agg_kernel


def _round_up(a, b):
    return (a + b - 1) // b * b


def kernel(x, edge_index, W1, b1, W2, b2):
    n, f_in = x.shape
    e = edge_index.shape[1]
    hid = W1.shape[1]
    c_out = W2.shape[1]
    assert n % NS == 0 and hid == 16

    r_pad = _round_up(-(-e // CH), NW * KR)
    e_pad = r_pad * CH
    pad = e_pad - e
    src2d = jnp.concatenate(
        [edge_index[0], jnp.zeros((pad,), jnp.int32)]).reshape(r_pad, CH)
    dst2d = jnp.concatenate(
        [edge_index[1], jnp.full((pad,), n, jnp.int32)]).reshape(r_pad, CH)

    n_deg = _round_up(n + 1, NS * 16)
    n_acc = _round_up(n + 1, NS * ZQ)

    deg_kernel = _make_deg_kernel(r_pad, n_deg)
    agg_kernel = _make_agg_kernel(n, r_pad, n_acc, hid)

    deg_parts = deg_kernel(dst2d).reshape(NC, n_deg)
    deg_col = (deg_parts[0, :n] + deg_parts[1, :n]).reshape(n, 1)

    bn = 2000
    grid = (n // bn,)

    def k2_body(deg_ref, x_ref, w1_ref, hs_ref, dinv_ref):
        dinv = lax.rsqrt(deg_ref[...] + 1.0)
        h = jnp.dot(x_ref[...], w1_ref[...],
                    preferred_element_type=jnp.float32)
        hs_ref[...] = h * dinv
        dinv_ref[...] = dinv

    hs, dinv = pl.pallas_call(
        k2_body,
        grid=grid,
        in_specs=[
            pl.BlockSpec((bn, 1), lambda i: (i, 0)),
            pl.BlockSpec((bn, f_in), lambda i: (i, 0)),
            pl.BlockSpec((f_in, hid), lambda i: (0, 0)),
        ],
        out_specs=[
            pl.BlockSpec((bn, hid), lambda i: (i, 0)),
            pl.BlockSpec((bn, 1), lambda i: (i, 0)),
        ],
        out_shape=[
            jax.ShapeDtypeStruct((n, hid), jnp.float32),
            jax.ShapeDtypeStruct((n, 1), jnp.float32),
        ],
    )(deg_col, x, W1)

    agg1 = agg_kernel(hs, src2d, dst2d)

    def k4_body(a_ref, hs_ref, dinv_ref, b1_ref, rs_ref):
        t = (a_ref[0] + a_ref[1] + hs_ref[...]) * dinv_ref[...] + b1_ref[...]
        rs_ref[...] = jnp.maximum(t, 0.0) * dinv_ref[...]

    rs = pl.pallas_call(
        k4_body,
        grid=grid,
        in_specs=[
            pl.BlockSpec((NC, bn, hid), lambda i: (0, i, 0)),
            pl.BlockSpec((bn, hid), lambda i: (i, 0)),
            pl.BlockSpec((bn, 1), lambda i: (i, 0)),
            pl.BlockSpec((1, hid), lambda i: (0, 0)),
        ],
        out_specs=pl.BlockSpec((bn, hid), lambda i: (i, 0)),
        out_shape=jax.ShapeDtypeStruct((n, hid), jnp.float32),
    )(agg1, hs, dinv, b1.reshape(1, hid))

    agg2 = agg_kernel(rs, src2d, dst2d)

    def k6_body(a_ref, rs_ref, dinv_ref, w2_ref, b2_ref, out_ref):
        t = (a_ref[0] + a_ref[1] + rs_ref[...]) * dinv_ref[...]
        out_ref[...] = jnp.dot(t, w2_ref[...],
                               preferred_element_type=jnp.float32) + b2_ref[...]

    out = pl.pallas_call(
        k6_body,
        grid=grid,
        in_specs=[
            pl.BlockSpec((NC, bn, hid), lambda i: (0, i, 0)),
            pl.BlockSpec((bn, hid), lambda i: (i, 0)),
            pl.BlockSpec((bn, 1), lambda i: (i, 0)),
            pl.BlockSpec((hid, c_out), lambda i: (0, 0)),
            pl.BlockSpec((1, c_out), lambda i: (0, 0)),
        ],
        out_specs=pl.BlockSpec((bn, c_out), lambda i: (i, 0)),
        out_shape=jax.ShapeDtypeStruct((n, c_out), jnp.float32),
    )(agg2, rs, dinv, W2, b2.reshape(1, c_out))

    return out
